# baseline (speedup 1.0000x reference)
import functools

import jax
import jax.numpy as jnp
from jax import lax
from jax.experimental import pallas as pl
from jax.experimental.pallas import tpu as pltpu
from jax.experimental.pallas import tpu_sc as plsc

_V = 1000001
_B = 16384
_D = 64
_NC = 2
_NS = 16
_NW = _NC * _NS
_NSLOT = 256
_MCAP = 1024
_WAVE = 8

_mesh = plsc.VectorSubcoreMesh(core_axis_name="c", subcore_axis_name="s")


@functools.partial(
    pl.kernel,
    mesh=_mesh,
    out_type=jax.ShapeDtypeStruct((_B, _D), jnp.float32),
    scratch_types=[
        pltpu.VMEM((_B,), jnp.int32),
        pltpu.VMEM((_MCAP,), jnp.int32),
        pltpu.VMEM((_MCAP,), jnp.int32),
        pltpu.VMEM((_NSLOT * 16,), jnp.int32),
        pltpu.VMEM((_NSLOT * 16,), jnp.int32),
        pltpu.VMEM((_WAVE, _D, 128), jnp.float32),
        pltpu.VMEM((64, 1, _D), jnp.float32),
        pltpu.SMEM((_NSLOT,), jnp.int32),
        pltpu.SMEM((8,), jnp.int32),
        pltpu.SemaphoreType.DMA,
        pltpu.SemaphoreType.DMA,
    ],
    compiler_params=pltpu.CompilerParams(needs_layout_passes=False),
)
def _embed_gather(labels_hbm, tablet_hbm, out_hbm, lab_v, ml_v, mp_v,
                  bl_v, bp_v, gbuf, rowring, cnt_s, misc_s, sem, osem):
    wid = lax.axis_index("s") * _NC + lax.axis_index("c")
    lanes = lax.iota(jnp.int32, 16)

    pltpu.sync_copy(labels_hbm, lab_v)

    def zero_cnt(i, carry):
        cnt_s[i] = 0
        return carry

    lax.fori_loop(0, _NSLOT, zero_cnt, 0)
    misc_s[0] = 0

    def filt(g, off):
        v = lab_v[pl.ds(g * 16, 16)]
        m = ((v >> 7) & 31) == wid
        mi = m.astype(jnp.int32)
        cs = plsc.cumsum(mi)
        dst = off + cs - mi
        plsc.store_scatter(ml_v, [dst], v, mask=m)
        plsc.store_scatter(mp_v, [dst], g * 16 + lanes, mask=m)
        return off + cs[15]

    n_match = lax.fori_loop(0, _B // 16, filt, 0)

    m0 = lanes == 0

    def bucket(g, carry):
        lv = ml_v[pl.ds(g * 16, 16)]
        pv = mp_v[pl.ds(g * 16, 16)]
        for b in range(16):
            i = g * 16 + b

            @pl.when(i < n_match)
            def _():
                l = lv[b]
                p = pv[b]
                slot = l >> 12
                c = cnt_s[slot]
                tgt = jnp.broadcast_to(slot * 16 + c, (16,))
                plsc.store_scatter(bl_v, [tgt],
                                   jnp.broadcast_to(l, (16,)), mask=m0)
                plsc.store_scatter(bp_v, [tgt],
                                   jnp.broadcast_to(p, (16,)), mask=m0)
                cnt_s[slot] = c + 1

        return carry

    lax.fori_loop(0, _MCAP // 16, bucket, 0)

    misc_s[1] = 0

    def wave(w, carry):
        counts = []
        for b in range(_WAVE):
            slot = w * _WAVE + b
            c = cnt_s[slot]
            counts.append((slot, c))

            @pl.when(c > 0)
            def _():
                off = pl.multiple_of((slot * 32 + wid) * 128, 128)
                pltpu.async_copy(
                    tablet_hbm.at[:, pl.ds(off, 128)], gbuf.at[b], sem
                )

        for b in range(_WAVE):
            slot, c = counts[b]

            @pl.when(c > 0)
            def _():
                pltpu.make_async_copy(
                    tablet_hbm.at[:, pl.ds(0, 128)], gbuf.at[0], sem
                ).wait()

        for b in range(_WAVE):
            slot, c = counts[b]
            blv = bl_v[pl.ds(slot * 16, 16)]
            bpv = bp_v[pl.ds(slot * 16, 16)]
            for b2 in range(16):

                @pl.when(b2 < c)
                def _():
                    l = blv[b2]
                    p = bpv[b2]
                    cvec = jnp.broadcast_to(l & 127, (16,))
                    nout = misc_s[1]
                    r = nout & 63

                    @pl.when(nout >= 64)
                    def _():
                        pltpu.make_async_copy(
                            rowring.at[0], out_hbm.at[pl.ds(0, 1)], osem
                        ).wait()

                    for k in range(_D // 16):
                        vals = plsc.load_gather(
                            gbuf.at[b], [k * 16 + lanes, cvec]
                        )
                        rowring[r, 0, pl.ds(k * 16, 16)] = vals
                    pltpu.async_copy(
                        rowring.at[r], out_hbm.at[pl.ds(p, 1)], osem
                    )
                    misc_s[1] = nout + 1

        return carry

    lax.fori_loop(0, _NSLOT // _WAVE, wave, 0)

    def drain(i, carry):
        @pl.when(i < jnp.minimum(misc_s[1], 64))
        def _():
            pltpu.make_async_copy(
                rowring.at[0], out_hbm.at[pl.ds(0, 1)], osem
            ).wait()

        return carry

    lax.fori_loop(0, 64, drain, 0)


def kernel(labels, table):
    return _embed_gather(labels.astype(jnp.int32), table.T)


# speedup vs baseline: 1.4193x; 1.4193x over previous
import functools

import jax
import jax.numpy as jnp
from jax import lax
from jax.experimental import pallas as pl
from jax.experimental.pallas import tpu as pltpu
from jax.experimental.pallas import tpu_sc as plsc

_V = 1000001
_B = 16384
_D = 64
_NC = 2
_NS = 16
_NW = _NC * _NS
_NSLOT = 256
_MCAP = 1024
_WAVE = 4

_mesh = plsc.VectorSubcoreMesh(core_axis_name="c", subcore_axis_name="s")


@functools.partial(
    pl.kernel,
    mesh=_mesh,
    out_type=jax.ShapeDtypeStruct((_B, _D), jnp.float32),
    scratch_types=[
        pltpu.VMEM((_B,), jnp.int32),
        pltpu.VMEM((_MCAP,), jnp.int32),
        pltpu.VMEM((_MCAP,), jnp.int32),
        pltpu.VMEM((_NSLOT * 16,), jnp.int32),
        pltpu.VMEM((_NSLOT * 16,), jnp.int32),
        pltpu.VMEM((2 * _WAVE, _D, 128), jnp.float32),
        pltpu.VMEM((64, 1, _D), jnp.float32),
        pltpu.SMEM((_NSLOT + 2 * _WAVE,), jnp.int32),
        pltpu.SMEM((8,), jnp.int32),
        pltpu.SemaphoreType.DMA,
        pltpu.SemaphoreType.DMA,
    ],
    compiler_params=pltpu.CompilerParams(needs_layout_passes=False),
)
def _embed_gather(labels_hbm, tablet_hbm, out_hbm, lab_v, ml_v, mp_v,
                  bl_v, bp_v, gbuf, rowring, cnt_s, misc_s, sem, osem):
    wid = lax.axis_index("s") * _NC + lax.axis_index("c")
    lanes = lax.iota(jnp.int32, 16)

    pltpu.sync_copy(labels_hbm, lab_v)

    def zero_cnt(i, carry):
        cnt_s[i] = 0
        return carry

    lax.fori_loop(0, _NSLOT + 2 * _WAVE, zero_cnt, 0)
    misc_s[0] = 0

    def filt(g, off):
        v = lab_v[pl.ds(g * 16, 16)]
        m = ((v >> 7) & 31) == wid
        mi = m.astype(jnp.int32)
        cs = plsc.cumsum(mi)
        dst = off + cs - mi
        plsc.store_scatter(ml_v, [dst], v, mask=m)
        plsc.store_scatter(mp_v, [dst], g * 16 + lanes, mask=m)
        return off + cs[15]

    n_match = lax.fori_loop(0, _B // 16, filt, 0)

    m0 = lanes == 0

    def bucket(g, carry):
        lv = ml_v[pl.ds(g * 16, 16)]
        pv = mp_v[pl.ds(g * 16, 16)]
        for b in range(16):
            i = g * 16 + b

            @pl.when(i < n_match)
            def _():
                l = lv[b]
                p = pv[b]
                slot = l >> 12
                c = cnt_s[slot]
                tgt = jnp.broadcast_to(slot * 16 + c, (16,))
                plsc.store_scatter(bl_v, [tgt],
                                   jnp.broadcast_to(l, (16,)), mask=m0)
                plsc.store_scatter(bp_v, [tgt],
                                   jnp.broadcast_to(p, (16,)), mask=m0)
                cnt_s[slot] = c + 1

        return carry

    lax.fori_loop(0, _MCAP // 16, bucket, 0)

    misc_s[1] = 0

    def fire_wave(w, set_base):
        for b in range(_WAVE):
            slot = w * _WAVE + b
            c = cnt_s[slot]

            @pl.when(c > 0)
            def _():
                off = pl.multiple_of((slot * 32 + wid) * 128, 128)
                pltpu.async_copy(
                    tablet_hbm.at[:, pl.ds(off, 128)],
                    gbuf.at[set_base + b],
                    sem,
                )

    def drain_extract_wave(w, set_base):
        for b in range(_WAVE):
            slot = w * _WAVE + b
            c = cnt_s[slot]

            @pl.when(c > 0)
            def _():
                pltpu.make_async_copy(
                    tablet_hbm.at[:, pl.ds(0, 128)], gbuf.at[0], sem
                ).wait()

        for b in range(_WAVE):
            slot = w * _WAVE + b
            c = cnt_s[slot]
            blv = bl_v[pl.ds(slot * 16, 16)]
            bpv = bp_v[pl.ds(slot * 16, 16)]
            for b2 in range(16):

                @pl.when(b2 < c)
                def _():
                    l = blv[b2]
                    p = bpv[b2]
                    cvec = jnp.broadcast_to(l & 127, (16,))
                    nout = misc_s[1]
                    r = nout & 63

                    @pl.when(nout >= 64)
                    def _():
                        pltpu.make_async_copy(
                            rowring.at[0], out_hbm.at[pl.ds(0, 1)], osem
                        ).wait()

                    for k in range(_D // 16):
                        vals = plsc.load_gather(
                            gbuf.at[set_base + b], [k * 16 + lanes, cvec]
                        )
                        rowring[r, 0, pl.ds(k * 16, 16)] = vals
                    pltpu.async_copy(
                        rowring.at[r], out_hbm.at[pl.ds(p, 1)], osem
                    )
                    misc_s[1] = nout + 1

    fire_wave(0, 0)

    def wave_pair(i, carry):
        fire_wave(2 * i + 1, _WAVE)
        drain_extract_wave(2 * i, 0)
        fire_wave(2 * i + 2, 0)
        drain_extract_wave(2 * i + 1, _WAVE)
        return carry

    lax.fori_loop(0, _NSLOT // (2 * _WAVE), wave_pair, 0)

    def drain(i, carry):
        @pl.when(i < jnp.minimum(misc_s[1], 64))
        def _():
            pltpu.make_async_copy(
                rowring.at[0], out_hbm.at[pl.ds(0, 1)], osem
            ).wait()

        return carry

    lax.fori_loop(0, 64, drain, 0)


def kernel(labels, table):
    return _embed_gather(labels.astype(jnp.int32), table.T)
